# Initial kernel scaffold; baseline (speedup 1.0000x reference)
import functools
import jax
import jax.numpy as jnp
from jax import lax
from jax.experimental import pallas as pl
from jax.experimental.pallas import tpu as pltpu
from jax.experimental.pallas import tpu_sc as plsc

C = 128
NW = 32
RPW = 160
R_PAD = NW * RPW
N_PTS = 64
OUT_W = C * N_PTS


def _roi_align_sc(table, boxes_flat, levels):
    mesh = plsc.VectorSubcoreMesh(core_axis_name="c", subcore_axis_name="s")

    @functools.partial(
        pl.kernel,
        out_type=jax.ShapeDtypeStruct((R_PAD, OUT_W), jnp.float32),
        mesh=mesh,
        scratch_types=[
            pltpu.VMEM((6 * RPW,), jnp.float32),
            pltpu.VMEM((RPW,), jnp.int32),
            pltpu.VMEM((16 * 24,), jnp.int32),
            pltpu.VMEM((16 * 24,), jnp.float32),
            pltpu.VMEM((4, 128), jnp.int32),
            pltpu.VMEM((512,), jnp.float32),
            pltpu.VMEM((128, C), jnp.float32),
            pltpu.VMEM((OUT_W,), jnp.float32),
            pltpu.SemaphoreType.DMA,
        ],
    )
    def k(table_hbm, boxes_hbm, lev_hbm, out_hbm,
          boxv, levv, itab, wtab, idxb, wb, rows, accb, sem):
        wid = lax.axis_index("s") * 2 + lax.axis_index("c")
        base_r = wid * RPW
        for a in range(6):
            pltpu.sync_copy(boxes_hbm.at[pl.ds(a * R_PAD + base_r, RPW)],
                            boxv.at[pl.ds(a * RPW, RPW)])
        pltpu.sync_copy(lev_hbm.at[pl.ds(base_r, RPW)], levv)

        lanes = lax.broadcasted_iota(jnp.int32, (16,), 0)
        oidx0 = lanes * N_PTS

        def blk_body(blk, carry):
            j0 = blk * 16
            lev = levv[pl.ds(j0, 16)]
            is1 = lev == 1
            scale = jnp.where(is1, jnp.float32(0.0625), jnp.float32(0.125))
            d_f = jnp.where(is1, jnp.float32(16.0), jnp.float32(32.0))
            d_i = jnp.where(is1, jnp.int32(16), jnp.int32(32))
            lbase = jnp.where(is1, jnp.int32(32768), jnp.int32(0))
            s_yz = jnp.where(is1, jnp.int32(256), jnp.int32(1024))
            s_z = jnp.where(is1, jnp.int32(16), jnp.int32(32))
            strides = (s_yz, s_z, jnp.full((16,), 1, jnp.int32))
            for a in range(3):
                st = boxv[pl.ds(a * RPW + j0, 16)] * scale
                en = boxv[pl.ds((a + 3) * RPW + j0, 16)] * scale
                binsz = jnp.maximum(en - st, 1.0) * 0.25
                abase = lbase if a == 0 else jnp.zeros((16,), jnp.int32)
                for b in range(4):
                    g = st + (b + 0.5) * binsz
                    g = jnp.clip(g, 0.0, d_f - 1.0)
                    lo = g.astype(jnp.int32)
                    w = g - lo.astype(jnp.float32)
                    hi = jnp.minimum(lo + 1, d_i - 1)
                    pos = lanes * 24 + (a * 8 + 2 * b)
                    plsc.store_scatter(itab, [pos], lo * strides[a] + abase)
                    plsc.store_scatter(itab, [pos + 1], hi * strides[a] + abase)
                    plsc.store_scatter(wtab, [pos], 1.0 - w)
                    plsc.store_scatter(wtab, [pos + 1], w)

            def roi_body(j, carry2):
                jbase = j * 24

                def tv_body(tv, c3):
                    t = tv * 16 + lanes
                    xs = jbase + ((t >> 7) & 3) * 2 + ((t >> 2) & 1)
                    ys = jbase + 8 + ((t >> 5) & 3) * 2 + ((t >> 1) & 1)
                    zs = jbase + 16 + ((t >> 3) & 3) * 2 + (t & 1)
                    iv = (plsc.load_gather(itab, [xs])
                          + plsc.load_gather(itab, [ys])
                          + plsc.load_gather(itab, [zs]))
                    wv = (plsc.load_gather(wtab, [xs])
                          * plsc.load_gather(wtab, [ys])
                          * plsc.load_gather(wtab, [zs]))
                    idxb[tv >> 3, pl.ds((tv & 7) * 16, 16)] = iv
                    wb[pl.ds(tv * 16, 16)] = wv
                    return c3

                lax.fori_loop(0, 32, tv_body, 0)

                for ch in range(4):
                    pltpu.async_copy(table_hbm.at[idxb.at[ch]], rows, sem).wait()

                    def pt_body(pp, c4, ch=ch):
                        acc = [jnp.zeros((16,), jnp.float32) for _ in range(8)]
                        for kk in range(8):
                            tloc = pp * 8 + kk
                            wsc = wb[ch * 128 + tloc]
                            wv = jnp.full((16,), wsc, jnp.float32)
                            for cc in range(8):
                                acc[cc] = acc[cc] + wv * rows[tloc,
                                                              pl.ds(cc * 16, 16)]
                        p = ch * 16 + pp
                        for cc in range(8):
                            plsc.store_scatter(
                                accb, [oidx0 + (cc * 16 * N_PTS + p)], acc[cc])
                        return c4

                    lax.fori_loop(0, 16, pt_body, 0)

                pltpu.sync_copy(accb, out_hbm.at[base_r + j0 + j])
                return carry2

            lax.fori_loop(0, 16, roi_body, 0)
            return carry

        lax.fori_loop(0, RPW // 16, blk_body, 0)

    return k(table, boxes_flat, levels)


def kernel(feat0, feat1, boxes):
    R = boxes.shape[0]
    f0 = feat0[0].transpose(1, 2, 3, 0).reshape(-1, C)
    f1 = feat1[0].transpose(1, 2, 3, 0).reshape(-1, C)
    table = jnp.concatenate([f0, f1], axis=0)
    vol = ((boxes[:, 3] - boxes[:, 0]) * (boxes[:, 4] - boxes[:, 1])
           * (boxes[:, 5] - boxes[:, 2]))
    s = jnp.power(jnp.maximum(vol, 1e-12), 1.0 / 3.0)
    lvl = jnp.floor(4.0 + jnp.log2(s / 160.0) + 1e-6)
    lev = (jnp.clip(lvl, 3.0, 4.0) - 3.0).astype(jnp.int32)
    boxes_t = jnp.zeros((6, R_PAD), jnp.float32).at[:, :R].set(boxes.T)
    lev_p = jnp.zeros((R_PAD,), jnp.int32).at[:R].set(lev)
    out = _roi_align_sc(table, boxes_t.reshape(-1), lev_p)
    return out[:R].reshape(R, C, 4, 4, 4)


# speedup vs baseline: 29.1597x; 29.1597x over previous
import functools
import jax
import jax.numpy as jnp
from jax import lax
from jax.experimental import pallas as pl
from jax.experimental.pallas import tpu as pltpu
from jax.experimental.pallas import tpu_sc as plsc

C = 128
NW = 32
RPW = 160
R_PAD = NW * RPW
N_PTS = 64
OUT_W = C * N_PTS


def _roi_align_sc(table, boxes_flat, levels):
    mesh = plsc.VectorSubcoreMesh(core_axis_name="c", subcore_axis_name="s")

    @functools.partial(
        pl.kernel,
        out_type=jax.ShapeDtypeStruct((R_PAD, OUT_W), jnp.float32),
        mesh=mesh,
        compiler_params=pltpu.CompilerParams(needs_layout_passes=False),
        scratch_types=[
            pltpu.VMEM((6 * RPW,), jnp.float32),
            pltpu.VMEM((RPW,), jnp.int32),
            pltpu.VMEM((16 * 24,), jnp.int32),
            pltpu.VMEM((16 * 24,), jnp.float32),
            pltpu.VMEM((4, 128), jnp.int32),
            pltpu.VMEM((512,), jnp.float32),
            pltpu.VMEM((128, C), jnp.float32),
            pltpu.VMEM((OUT_W,), jnp.float32),
            pltpu.SemaphoreType.DMA,
        ],
    )
    def k(table_hbm, boxes_hbm, lev_hbm, out_hbm,
          boxv, levv, itab, wtab, idxb, wb, rows, accb, sem):
        wid = lax.axis_index("s") * 2 + lax.axis_index("c")
        base_r = wid * RPW
        for a in range(6):
            pltpu.sync_copy(boxes_hbm.at[pl.ds(a * R_PAD + base_r, RPW)],
                            boxv.at[pl.ds(a * RPW, RPW)])
        pltpu.sync_copy(lev_hbm.at[pl.ds(base_r, RPW)], levv)

        lanes = lax.broadcasted_iota(jnp.int32, (16,), 0)
        oidx0 = lanes * N_PTS

        def blk_body(blk, carry):
            j0 = blk * 16
            lev = levv[pl.ds(j0, 16)]
            is1 = lev == 1
            scale = jnp.where(is1, jnp.float32(0.0625), jnp.float32(0.125))
            d_f = jnp.where(is1, jnp.float32(16.0), jnp.float32(32.0))
            d_i = jnp.where(is1, jnp.int32(16), jnp.int32(32))
            lbase = jnp.where(is1, jnp.int32(32768), jnp.int32(0))
            s_yz = jnp.where(is1, jnp.int32(256), jnp.int32(1024))
            s_z = jnp.where(is1, jnp.int32(16), jnp.int32(32))
            strides = (s_yz, s_z, jnp.full((16,), 1, jnp.int32))
            for a in range(3):
                st = boxv[pl.ds(a * RPW + j0, 16)] * scale
                en = boxv[pl.ds((a + 3) * RPW + j0, 16)] * scale
                binsz = jnp.maximum(en - st, 1.0) * 0.25
                abase = lbase if a == 0 else jnp.zeros((16,), jnp.int32)
                for b in range(4):
                    g = st + (b + 0.5) * binsz
                    g = jnp.clip(g, 0.0, d_f - 1.0)
                    lo = g.astype(jnp.int32)
                    w = g - lo.astype(jnp.float32)
                    hi = jnp.minimum(lo + 1, d_i - 1)
                    pos = lanes * 24 + (a * 8 + 2 * b)
                    plsc.store_scatter(itab, [pos], lo * strides[a] + abase)
                    plsc.store_scatter(itab, [pos + 1], hi * strides[a] + abase)
                    plsc.store_scatter(wtab, [pos], 1.0 - w)
                    plsc.store_scatter(wtab, [pos + 1], w)

            def roi_body(j, carry2):
                jbase = j * 24

                def tv_body(tv, c3):
                    t = tv * 16 + lanes
                    xs = jbase + ((t >> 7) & 3) * 2 + ((t >> 2) & 1)
                    ys = jbase + 8 + ((t >> 5) & 3) * 2 + ((t >> 1) & 1)
                    zs = jbase + 16 + ((t >> 3) & 3) * 2 + (t & 1)
                    iv = (plsc.load_gather(itab, [xs])
                          + plsc.load_gather(itab, [ys])
                          + plsc.load_gather(itab, [zs]))
                    wv = (plsc.load_gather(wtab, [xs])
                          * plsc.load_gather(wtab, [ys])
                          * plsc.load_gather(wtab, [zs]))
                    idxb[tv >> 3, pl.ds((tv & 7) * 16, 16)] = iv
                    wb[pl.ds(tv * 16, 16)] = wv
                    return c3

                lax.fori_loop(0, 32, tv_body, 0)

                for ch in range(4):
                    pltpu.async_copy(table_hbm.at[idxb.at[ch]], rows, sem).wait()

                    def pair_body(pr, c4, ch=ch):
                        wv16 = wb[pl.ds(ch * 128 + pr * 16, 16)]
                        for half in range(2):
                            acc = [jnp.zeros((16,), jnp.float32)
                                   for _ in range(8)]
                            for kk in range(8):
                                tloc = pr * 16 + half * 8 + kk
                                wv = jnp.full((16,), wv16[half * 8 + kk],
                                              jnp.float32)
                                for cc in range(8):
                                    acc[cc] = acc[cc] + wv * rows[
                                        tloc, pl.ds(cc * 16, 16)]
                            p = ch * 16 + pr * 2 + half
                            for cc in range(8):
                                plsc.store_scatter(
                                    accb, [oidx0 + (cc * 16 * N_PTS + p)],
                                    acc[cc])
                        return c4

                    lax.fori_loop(0, 8, pair_body, 0)

                pltpu.sync_copy(accb, out_hbm.at[base_r + j0 + j])
                return carry2

            lax.fori_loop(0, 16, roi_body, 0)
            return carry

        lax.fori_loop(0, RPW // 16, blk_body, 0)

    return k(table, boxes_flat, levels)


def kernel(feat0, feat1, boxes):
    R = boxes.shape[0]
    f0 = feat0[0].transpose(1, 2, 3, 0).reshape(-1, C)
    f1 = feat1[0].transpose(1, 2, 3, 0).reshape(-1, C)
    table = jnp.concatenate([f0, f1], axis=0)
    vol = ((boxes[:, 3] - boxes[:, 0]) * (boxes[:, 4] - boxes[:, 1])
           * (boxes[:, 5] - boxes[:, 2]))
    s = jnp.power(jnp.maximum(vol, 1e-12), 1.0 / 3.0)
    lvl = jnp.floor(4.0 + jnp.log2(s / 160.0) + 1e-6)
    lev = (jnp.clip(lvl, 3.0, 4.0) - 3.0).astype(jnp.int32)
    boxes_t = jnp.zeros((6, R_PAD), jnp.float32).at[:, :R].set(boxes.T)
    lev_p = jnp.zeros((R_PAD,), jnp.int32).at[:R].set(lev)
    out = _roi_align_sc(table, boxes_t.reshape(-1), lev_p)
    return out[:R].reshape(R, C, 4, 4, 4)


# speedup vs baseline: 35.5288x; 1.2184x over previous
import functools
import jax
import jax.numpy as jnp
from jax import lax
from jax.experimental import pallas as pl
from jax.experimental.pallas import tpu as pltpu
from jax.experimental.pallas import tpu_sc as plsc

C = 128
NW = 32
RPW = 160
R_PAD = NW * RPW
N_PTS = 64
OUT_W = C * N_PTS


def _roi_align_sc(table, boxes_flat, levels):
    mesh = plsc.VectorSubcoreMesh(core_axis_name="c", subcore_axis_name="s")

    @functools.partial(
        pl.kernel,
        out_type=jax.ShapeDtypeStruct((R_PAD, OUT_W), jnp.float32),
        mesh=mesh,
        compiler_params=pltpu.CompilerParams(needs_layout_passes=False),
        scratch_types=[
            pltpu.VMEM((6 * RPW,), jnp.float32),
            pltpu.VMEM((RPW,), jnp.int32),
            pltpu.VMEM((16 * 24,), jnp.int32),
            pltpu.VMEM((16 * 24,), jnp.float32),
            pltpu.VMEM((4, 128), jnp.int32),
            pltpu.VMEM((512,), jnp.float32),
            pltpu.VMEM((4, 128, C), jnp.float32),
            pltpu.VMEM((2 * OUT_W,), jnp.float32),
            pltpu.SemaphoreType.DMA,
            pltpu.SemaphoreType.DMA,
            pltpu.SemaphoreType.DMA,
            pltpu.SemaphoreType.DMA,
            pltpu.SemaphoreType.DMA,
            pltpu.SemaphoreType.DMA,
        ],
    )
    def k(table_hbm, boxes_hbm, lev_hbm, out_hbm,
          boxv, levv, itab, wtab, idxb, wb, rows, accb,
          g0, g1, g2, g3, o0, o1):
        gsem = (g0, g1, g2, g3)
        osem = (o0, o1)
        wid = lax.axis_index("s") * 2 + lax.axis_index("c")
        base_r = wid * RPW
        for a in range(6):
            pltpu.sync_copy(boxes_hbm.at[pl.ds(a * R_PAD + base_r, RPW)],
                            boxv.at[pl.ds(a * RPW, RPW)])
        pltpu.sync_copy(lev_hbm.at[pl.ds(base_r, RPW)], levv)

        lanes = lax.broadcasted_iota(jnp.int32, (16,), 0)
        oidx0 = lanes * N_PTS

        def blk_body(blk, carry):
            j0 = blk * 16
            lev = levv[pl.ds(j0, 16)]
            is1 = lev == 1
            scale = jnp.where(is1, jnp.float32(0.0625), jnp.float32(0.125))
            d_f = jnp.where(is1, jnp.float32(16.0), jnp.float32(32.0))
            d_i = jnp.where(is1, jnp.int32(16), jnp.int32(32))
            lbase = jnp.where(is1, jnp.int32(32768), jnp.int32(0))
            s_yz = jnp.where(is1, jnp.int32(256), jnp.int32(1024))
            s_z = jnp.where(is1, jnp.int32(16), jnp.int32(32))
            strides = (s_yz, s_z, jnp.full((16,), 1, jnp.int32))
            for a in range(3):
                st = boxv[pl.ds(a * RPW + j0, 16)] * scale
                en = boxv[pl.ds((a + 3) * RPW + j0, 16)] * scale
                binsz = jnp.maximum(en - st, 1.0) * 0.25
                abase = lbase if a == 0 else jnp.zeros((16,), jnp.int32)
                for b in range(4):
                    g = st + (b + 0.5) * binsz
                    g = jnp.clip(g, 0.0, d_f - 1.0)
                    lo = g.astype(jnp.int32)
                    w = g - lo.astype(jnp.float32)
                    hi = jnp.minimum(lo + 1, d_i - 1)
                    pos = lanes * 24 + (a * 8 + 2 * b)
                    plsc.store_scatter(itab, [pos], lo * strides[a] + abase)
                    plsc.store_scatter(itab, [pos + 1], hi * strides[a] + abase)
                    plsc.store_scatter(wtab, [pos], 1.0 - w)
                    plsc.store_scatter(wtab, [pos + 1], w)

            def pair_body(pr, carry2):
                for half in range(2):
                    j = pr * 2 + half
                    jbase = j * 24

                    def tv_body(tv, c3):
                        t = tv * 16 + lanes
                        xs = jbase + ((t >> 7) & 3) * 2 + ((t >> 2) & 1)
                        ys = jbase + 8 + ((t >> 5) & 3) * 2 + ((t >> 1) & 1)
                        zs = jbase + 16 + ((t >> 3) & 3) * 2 + (t & 1)
                        iv = (plsc.load_gather(itab, [xs])
                              + plsc.load_gather(itab, [ys])
                              + plsc.load_gather(itab, [zs]))
                        wv = (plsc.load_gather(wtab, [xs])
                              * plsc.load_gather(wtab, [ys])
                              * plsc.load_gather(wtab, [zs]))
                        idxb[tv >> 3, pl.ds((tv & 7) * 16, 16)] = iv
                        wb[pl.ds(tv * 16, 16)] = wv
                        return c3

                    lax.fori_loop(0, 32, tv_body, 0)

                    cps = [pltpu.async_copy(table_hbm.at[idxb.at[ch]],
                                            rows.at[ch], gsem[ch])
                           for ch in range(4)]

                    @pl.when(jnp.logical_or(blk > 0, pr > 0))
                    def _():
                        pltpu.make_async_copy(
                            out_hbm.at[0],
                            accb.at[pl.ds(half * OUT_W, OUT_W)],
                            osem[half]).wait()

                    abase_o = half * OUT_W

                    for ch in range(4):
                        cps[ch].wait()

                        def pair_pts(prp, c4, ch=ch):
                            wv16 = wb[pl.ds(ch * 128 + prp * 16, 16)]
                            for h2 in range(2):
                                acc = [jnp.zeros((16,), jnp.float32)
                                       for _ in range(8)]
                                for kk in range(8):
                                    tloc = prp * 16 + h2 * 8 + kk
                                    wv = jnp.full((16,), wv16[h2 * 8 + kk],
                                                  jnp.float32)
                                    for cc in range(8):
                                        acc[cc] = acc[cc] + wv * rows[
                                            ch, tloc, pl.ds(cc * 16, 16)]
                                p = ch * 16 + prp * 2 + h2
                                for cc in range(8):
                                    plsc.store_scatter(
                                        accb,
                                        [oidx0 + (abase_o + cc * 16 * N_PTS + p)],
                                        acc[cc])
                            return c4

                        lax.fori_loop(0, 8, pair_pts, 0)

                    pltpu.async_copy(accb.at[pl.ds(half * OUT_W, OUT_W)],
                                     out_hbm.at[base_r + j0 + j], osem[half])
                return carry2

            lax.fori_loop(0, 8, pair_body, 0)
            return carry

        lax.fori_loop(0, RPW // 16, blk_body, 0)

        for half in range(2):
            pltpu.make_async_copy(out_hbm.at[0],
                                  accb.at[pl.ds(half * OUT_W, OUT_W)],
                                  osem[half]).wait()

    return k(table, boxes_flat, levels)


def kernel(feat0, feat1, boxes):
    R = boxes.shape[0]
    f0 = feat0[0].transpose(1, 2, 3, 0).reshape(-1, C)
    f1 = feat1[0].transpose(1, 2, 3, 0).reshape(-1, C)
    table = jnp.concatenate([f0, f1], axis=0)
    vol = ((boxes[:, 3] - boxes[:, 0]) * (boxes[:, 4] - boxes[:, 1])
           * (boxes[:, 5] - boxes[:, 2]))
    s = jnp.power(jnp.maximum(vol, 1e-12), 1.0 / 3.0)
    lvl = jnp.floor(4.0 + jnp.log2(s / 160.0) + 1e-6)
    lev = (jnp.clip(lvl, 3.0, 4.0) - 3.0).astype(jnp.int32)
    boxes_t = jnp.zeros((6, R_PAD), jnp.float32).at[:, :R].set(boxes.T)
    lev_p = jnp.zeros((R_PAD,), jnp.int32).at[:R].set(lev)
    out = _roi_align_sc(table, boxes_t.reshape(-1), lev_p)
    return out[:R].reshape(R, C, 4, 4, 4)


# speedup vs baseline: 46.9698x; 1.3220x over previous
import functools
import jax
import jax.numpy as jnp
from jax import lax
from jax.experimental import pallas as pl
from jax.experimental.pallas import tpu as pltpu
from jax.experimental.pallas import tpu_sc as plsc

C = 128
NW = 32
RPW = 160
R_PAD = NW * RPW
N_PTS = 64
OUT_W = C * N_PTS


def _roi_align_sc(table, boxes_flat, levels):
    mesh = plsc.VectorSubcoreMesh(core_axis_name="c", subcore_axis_name="s")

    @functools.partial(
        pl.kernel,
        out_type=jax.ShapeDtypeStruct((R_PAD, OUT_W), jnp.float32),
        mesh=mesh,
        compiler_params=pltpu.CompilerParams(needs_layout_passes=False),
        scratch_types=[
            pltpu.VMEM((6 * RPW,), jnp.float32),
            pltpu.VMEM((RPW,), jnp.int32),
            pltpu.VMEM((RPW * 24,), jnp.int32),
            pltpu.VMEM((RPW * 24,), jnp.float32),
            pltpu.VMEM((2, 4, 128), jnp.int32),
            pltpu.VMEM((2, 512), jnp.float32),
            pltpu.VMEM((4, 128, C), jnp.float32),
            pltpu.VMEM((2 * OUT_W,), jnp.float32),
            pltpu.SemaphoreType.DMA,
            pltpu.SemaphoreType.DMA,
            pltpu.SemaphoreType.DMA,
            pltpu.SemaphoreType.DMA,
            pltpu.SemaphoreType.DMA,
            pltpu.SemaphoreType.DMA,
        ],
    )
    def k(table_hbm, boxes_hbm, lev_hbm, out_hbm,
          boxv, levv, itab, wtab, idxb, wb, rows, accb,
          g0, g1, g2, g3, o0, o1):
        gsem = (g0, g1, g2, g3)
        osem = (o0, o1)
        wid = lax.axis_index("s") * 2 + lax.axis_index("c")
        base_r = wid * RPW
        for a in range(6):
            pltpu.sync_copy(boxes_hbm.at[pl.ds(a * R_PAD + base_r, RPW)],
                            boxv.at[pl.ds(a * RPW, RPW)])
        pltpu.sync_copy(lev_hbm.at[pl.ds(base_r, RPW)], levv)

        lanes = lax.broadcasted_iota(jnp.int32, (16,), 0)
        oidx0 = lanes * N_PTS

        def blk_body(blk, carry):
            j0 = blk * 16
            lev = levv[pl.ds(j0, 16)]
            is1 = lev == 1
            scale = jnp.where(is1, jnp.float32(0.0625), jnp.float32(0.125))
            d_f = jnp.where(is1, jnp.float32(16.0), jnp.float32(32.0))
            d_i = jnp.where(is1, jnp.int32(16), jnp.int32(32))
            lbase = jnp.where(is1, jnp.int32(32768), jnp.int32(0))
            s_yz = jnp.where(is1, jnp.int32(256), jnp.int32(1024))
            s_z = jnp.where(is1, jnp.int32(16), jnp.int32(32))
            strides = (s_yz, s_z, jnp.full((16,), 1, jnp.int32))
            for a in range(3):
                st = boxv[pl.ds(a * RPW + j0, 16)] * scale
                en = boxv[pl.ds((a + 3) * RPW + j0, 16)] * scale
                binsz = jnp.maximum(en - st, 1.0) * 0.25
                abase = lbase if a == 0 else jnp.zeros((16,), jnp.int32)
                for b in range(4):
                    g = st + (b + 0.5) * binsz
                    g = jnp.clip(g, 0.0, d_f - 1.0)
                    lo = g.astype(jnp.int32)
                    w = g - lo.astype(jnp.float32)
                    hi = jnp.minimum(lo + 1, d_i - 1)
                    pos = (j0 + lanes) * 24 + (a * 8 + 2 * b)
                    plsc.store_scatter(itab, [pos], lo * strides[a] + abase)
                    plsc.store_scatter(itab, [pos + 1], hi * strides[a] + abase)
                    plsc.store_scatter(wtab, [pos], 1.0 - w)
                    plsc.store_scatter(wtab, [pos + 1], w)
            return carry

        lax.fori_loop(0, RPW // 16, blk_body, 0)

        def build_idx(j, bf):
            jbase = j * 24

            def tv_body(tv, c3):
                t = tv * 16 + lanes
                xs = jbase + ((t >> 7) & 3) * 2 + ((t >> 2) & 1)
                ys = jbase + 8 + ((t >> 5) & 3) * 2 + ((t >> 1) & 1)
                zs = jbase + 16 + ((t >> 3) & 3) * 2 + (t & 1)
                iv = (plsc.load_gather(itab, [xs])
                      + plsc.load_gather(itab, [ys])
                      + plsc.load_gather(itab, [zs]))
                wv = (plsc.load_gather(wtab, [xs])
                      * plsc.load_gather(wtab, [ys])
                      * plsc.load_gather(wtab, [zs]))
                idxb[bf, tv >> 3, pl.ds((tv & 7) * 16, 16)] = iv
                wb[bf, pl.ds(tv * 16, 16)] = wv
                return c3

            lax.fori_loop(0, 32, tv_body, 0)

        build_idx(0, 0)
        for ch in range(4):
            pltpu.async_copy(table_hbm.at[idxb.at[0, ch]],
                             rows.at[ch], gsem[ch])

        def roi_pair_body(pr, carry):
            for cur in range(2):
                nxt = 1 - cur
                j = pr * 2 + cur
                last = (pr == RPW // 2 - 1) if cur == 1 else None

                if cur == 0:
                    build_idx(j + 1, nxt)
                else:
                    @pl.when(jnp.logical_not(last))
                    def _():
                        build_idx(j + 1, nxt)

                @pl.when(j > 1)
                def _():
                    pltpu.make_async_copy(
                        out_hbm.at[0],
                        accb.at[pl.ds(cur * OUT_W, OUT_W)],
                        osem[cur]).wait()

                abase_o = cur * OUT_W

                for ch in range(4):
                    pltpu.make_async_copy(table_hbm.at[idxb.at[cur, ch]],
                                          rows.at[ch], gsem[ch]).wait()

                    def pair_pts(prp, c4, ch=ch, cur=cur, abase_o=abase_o):
                        wv16 = wb[cur, pl.ds(ch * 128 + prp * 16, 16)]
                        for h2 in range(2):
                            acc = [jnp.zeros((16,), jnp.float32)
                                   for _ in range(8)]
                            for kk in range(8):
                                tloc = prp * 16 + h2 * 8 + kk
                                wv = jnp.full((16,), wv16[h2 * 8 + kk],
                                              jnp.float32)
                                for cc in range(8):
                                    acc[cc] = acc[cc] + wv * rows[
                                        ch, tloc, pl.ds(cc * 16, 16)]
                            p = ch * 16 + prp * 2 + h2
                            for cc in range(8):
                                plsc.store_scatter(
                                    accb,
                                    [oidx0 + (abase_o + cc * 16 * N_PTS + p)],
                                    acc[cc])
                        return c4

                    lax.fori_loop(0, 8, pair_pts, 0)

                    if cur == 0:
                        pltpu.async_copy(table_hbm.at[idxb.at[nxt, ch]],
                                         rows.at[ch], gsem[ch])
                    else:
                        @pl.when(jnp.logical_not(last))
                        def _(ch=ch, nxt=nxt):
                            pltpu.async_copy(table_hbm.at[idxb.at[nxt, ch]],
                                             rows.at[ch], gsem[ch])

                pltpu.async_copy(accb.at[pl.ds(cur * OUT_W, OUT_W)],
                                 out_hbm.at[base_r + j], osem[cur])
            return carry

        lax.fori_loop(0, RPW // 2, roi_pair_body, 0)

        for half in range(2):
            pltpu.make_async_copy(out_hbm.at[0],
                                  accb.at[pl.ds(half * OUT_W, OUT_W)],
                                  osem[half]).wait()

    return k(table, boxes_flat, levels)


def kernel(feat0, feat1, boxes):
    R = boxes.shape[0]
    f0 = feat0[0].transpose(1, 2, 3, 0).reshape(-1, C)
    f1 = feat1[0].transpose(1, 2, 3, 0).reshape(-1, C)
    table = jnp.concatenate([f0, f1], axis=0)
    vol = ((boxes[:, 3] - boxes[:, 0]) * (boxes[:, 4] - boxes[:, 1])
           * (boxes[:, 5] - boxes[:, 2]))
    s = jnp.power(jnp.maximum(vol, 1e-12), 1.0 / 3.0)
    lvl = jnp.floor(4.0 + jnp.log2(s / 160.0) + 1e-6)
    lev = (jnp.clip(lvl, 3.0, 4.0) - 3.0).astype(jnp.int32)
    boxes_t = jnp.zeros((6, R_PAD), jnp.float32).at[:, :R].set(boxes.T)
    lev_p = jnp.zeros((R_PAD,), jnp.int32).at[:R].set(lev)
    out = _roi_align_sc(table, boxes_t.reshape(-1), lev_p)
    return out[:R].reshape(R, C, 4, 4, 4)
